# Initial kernel scaffold; baseline (speedup 1.0000x reference)
import jax
import jax.numpy as jnp
from jax.experimental import pallas as pl


def kernel(prefix_ids, key_table, value_table):
    raise NotImplementedError("write your pallas kernel here")


# speedup vs baseline: 1.3051x; 1.3051x over previous
import functools

import jax
import jax.numpy as jnp
from jax import lax
from jax.experimental import pallas as pl
from jax.experimental.pallas import tpu as pltpu
from jax.experimental.pallas import tpu_sc as plsc

B, S, H = 64, 128, 2048
N = B * S
NC, NS = 2, 16
NW = NC * NS
ROWS_PER_W = N // NW
C = 16
NCHUNK = ROWS_PER_W // C

_mesh = plsc.VectorSubcoreMesh(core_axis_name="c", subcore_axis_name="s")


@functools.partial(
    pl.kernel,
    mesh=_mesh,
    out_type=(
        jax.ShapeDtypeStruct((N, H), jnp.float32),
        jax.ShapeDtypeStruct((N, H), jnp.float32),
    ),
    scratch_types=[
        pltpu.VMEM((NCHUNK, C), jnp.int32),
        pltpu.VMEM((C, H), jnp.float32),
        pltpu.VMEM((C, H), jnp.float32),
        pltpu.SemaphoreType.DMA,
        pltpu.SemaphoreType.DMA,
    ],
)
def _gather_kernel(ids_hbm, ktab_hbm, vtab_hbm, kout_hbm, vout_hbm,
                   idx_v, buf0, buf1, sem0, sem1):
    wid = lax.axis_index("s") * NC + lax.axis_index("c")
    base = wid * ROWS_PER_W
    pltpu.sync_copy(ids_hbm.at[wid], idx_v)
    for tab, out, buf, sem in ((ktab_hbm, kout_hbm, buf0, sem0),
                               (vtab_hbm, vout_hbm, buf1, sem1)):
        for j in range(NCHUNK):
            pltpu.async_copy(tab.at[idx_v.at[j]], buf, sem).wait()
            pltpu.sync_copy(buf, out.at[pl.ds(base + j * C, C)])


def kernel(prefix_ids, key_table, value_table):
    ids = prefix_ids.reshape(NW, NCHUNK, C)
    k, v = _gather_kernel(ids, key_table, value_table)
    return k.reshape(B, S, H), v.reshape(B, S, H)


# speedup vs baseline: 1.4048x; 1.0764x over previous
import functools

import jax
import jax.numpy as jnp
from jax import lax
from jax.experimental import pallas as pl
from jax.experimental.pallas import tpu as pltpu
from jax.experimental.pallas import tpu_sc as plsc

B, S, H = 64, 128, 2048
N = B * S
NC, NS = 2, 16
NW = NC * NS
ROWS_PER_W = N // NW
C = 16
NCHUNK = ROWS_PER_W // C

_mesh = plsc.VectorSubcoreMesh(core_axis_name="c", subcore_axis_name="s")


@functools.partial(
    pl.kernel,
    mesh=_mesh,
    out_type=(
        jax.ShapeDtypeStruct((N, H), jnp.float32),
        jax.ShapeDtypeStruct((N, H), jnp.float32),
    ),
    scratch_types=[
        pltpu.VMEM((NCHUNK, C), jnp.int32),
        pltpu.VMEM((C, H), jnp.float32),
        pltpu.VMEM((C, H), jnp.float32),
        pltpu.SemaphoreType.DMA,
        pltpu.SemaphoreType.DMA,
        pltpu.SemaphoreType.DMA,
        pltpu.SemaphoreType.DMA,
    ],
)
def _gather_kernel(ids_hbm, ktab_hbm, vtab_hbm, kout_hbm, vout_hbm,
                   idx_v, buf0, buf1, gsem0, gsem1, ssem0, ssem1):
    wid = lax.axis_index("s") * NC + lax.axis_index("c")
    base = wid * ROWS_PER_W
    pltpu.sync_copy(ids_hbm.at[wid], idx_v)
    bufs, gsems, ssems = (buf0, buf1), (gsem0, gsem1), (ssem0, ssem1)
    jobs = [(tab, out, j)
            for tab, out in ((ktab_hbm, kout_hbm), (vtab_hbm, vout_hbm))
            for j in range(NCHUNK)]
    scatters = [None, None]
    for t, (tab, out, j) in enumerate(jobs):
        bi = t % 2
        if scatters[bi] is not None:
            scatters[bi].wait()
        g = pltpu.async_copy(tab.at[idx_v.at[j]], bufs[bi], gsems[bi])
        g.wait()
        scatters[bi] = pltpu.async_copy(
            bufs[bi], out.at[pl.ds(base + j * C, C)], ssems[bi])
    scatters[0].wait()
    scatters[1].wait()


def kernel(prefix_ids, key_table, value_table):
    ids = prefix_ids.reshape(NW, NCHUNK, C)
    k, v = _gather_kernel(ids, key_table, value_table)
    return k.reshape(B, S, H), v.reshape(B, S, H)
